# baseline (speedup 1.0000x reference)
import jax
import jax.numpy as jnp
import numpy as np
from jax import lax
from jax.experimental import pallas as pl
from jax.experimental.pallas import tpu as pltpu
from jax.experimental.pallas import tpu_sc as plsc

_VOCAB = 1000000
_HIDDEN = 64
_SEQ = 200
_BATCH = 4096

_NC = 2
_NS = 16
_NW = _NC * _NS
_SEQ_PER_W = _BATCH // _NW

_G0 = 128
_G1 = _SEQ - _G0
_UNROLL = 4
_IDXN = 208

_RB = 16384


_SUB = 128

_A_NP = np.zeros((2, _SUB // 2, _SUB), np.float32)
_A_NP[0, np.arange(_SUB // 2), 2 * np.arange(_SUB // 2)] = 1.0
_A_NP[1, np.arange(_SUB // 2), 2 * np.arange(_SUB // 2) + 1] = 1.0


def _repack_body(a_ref, in_ref, out_ref):
    x = in_ref[...]
    a = a_ref[...]
    a_e = a[: _SUB // 2]
    a_o = a[_SUB // 2:]
    dn = (((1,), (1,)), ((), ()))
    parts = []
    for k in range(_RB // _SUB):
        xs = x[:, k * _SUB:(k + 1) * _SUB]
        e = jax.lax.dot_general(a_e, xs, dn,
                                precision=jax.lax.Precision.HIGHEST,
                                preferred_element_type=jnp.float32)
        o = jax.lax.dot_general(a_o, xs, dn,
                                precision=jax.lax.Precision.HIGHEST,
                                preferred_element_type=jnp.float32)
        parts.append(jnp.concatenate([e, o], axis=1))
    out_ref[...] = jnp.concatenate(parts, axis=0)


def _repack(emb_t):
    grid = (pl.cdiv(_VOCAB, _RB),)
    return pl.pallas_call(
        _repack_body,
        grid=grid,
        in_specs=[pl.BlockSpec((_SUB, _SUB), lambda j: (0, 0)),
                  pl.BlockSpec((_HIDDEN, _RB), lambda j: (0, j))],
        out_specs=pl.BlockSpec((_RB // 2, 128), lambda j: (j, 0)),
        out_shape=jax.ShapeDtypeStruct((_VOCAB * _HIDDEN // 128, 128),
                                       jnp.float32),
    )(jnp.asarray(_A_NP.reshape(_SUB, _SUB)), emb_t)


def _body(tok_hbm, emb_hbm, pos_hbm, out_hbm, pos_v,
          idxr0, idxr1, rows0, rows1, wb0, wb1,
          si0, si1, sg0, sg1, so0, so1):
    c = lax.axis_index("c")
    s = lax.axis_index("s")
    wid = s * _NC + c

    idxr = (idxr0, idxr1)
    rows = (rows0, rows1)
    wb = (wb0, wb1)
    si = (si0, si1)
    sg = (sg0, sg1)
    so = (so0, so1)

    def fire_idx(i, b):
        base = (wid * _SEQ_PER_W + i) * _SEQ
        pltpu.async_copy(
            tok_hbm.at[pl.ds(base, _SEQ)], idxr[b].at[pl.ds(0, _SEQ)], si[b])

    def wait_idx(b):
        pltpu.make_async_copy(
            tok_hbm.at[pl.ds(0, _SEQ)], idxr[b].at[pl.ds(0, _SEQ)],
            si[b]).wait()

    def fire_gather(b):
        pltpu.async_copy(
            emb_hbm.at[idxr[b].at[pl.ds(0, _G0)]], rows[b].at[pl.ds(0, _G0)],
            sg[b])
        pltpu.async_copy(
            emb_hbm.at[idxr[b].at[pl.ds(_G0, _G1)]],
            rows[b].at[pl.ds(_G0, _G1)], sg[b])

    def wait_gather(b):
        pltpu.make_async_copy(
            emb_hbm.at[pl.ds(0, _SEQ)], rows[b], sg[b]).wait()

    def fire_wb(i, b):
        seq = wid * _SEQ_PER_W + i
        pltpu.async_copy(wb[b], out_hbm.at[seq, :, pl.ds(0, _HIDDEN)],
                         so[b])

    def wait_wb(b):
        pltpu.make_async_copy(
            out_hbm.at[0, :, pl.ds(0, _HIDDEN)], wb[b], so[b]).wait()

    pltpu.sync_copy(pos_hbm, pos_v)

    fire_idx(0, 0)
    fire_idx(1, 1)
    wait_idx(0)
    fire_gather(0)

    def seq_body(i, carry):
        b = lax.rem(i, 2)

        def for_buf(bb):
            nb = 1 - bb

            @pl.when(i + 1 < _SEQ_PER_W)
            def _():
                wait_idx(nb)
                fire_gather(nb)

            wait_gather(bb)

            @pl.when(i >= 2)
            def _():
                wait_wb(bb)

            def l_body(j, c2):
                for u in range(_UNROLL):
                    l = j * _UNROLL + u
                    for g in range(_HIDDEN // 16):
                        sl = pl.ds(g * 16, 16)
                        wb[bb][l, sl] = rows[bb][l, sl] + pos_v[l, sl]
                return c2

            lax.fori_loop(0, _SEQ // _UNROLL, l_body, 0)

            fire_wb(i, bb)

            @pl.when(i + 2 < _SEQ_PER_W)
            def _():
                fire_idx(i + 2, bb)

        @pl.when(b == 0)
        def _():
            for_buf(0)

        @pl.when(b == 1)
        def _():
            for_buf(1)

        return carry

    lax.fori_loop(0, _SEQ_PER_W, seq_body, 0)

    wait_wb((_SEQ_PER_W - 1) % 2)


def _emb_lookup(tokens_flat, emb_pad, pos_table):
    kfn = pl.kernel(
        _body,
        mesh=plsc.VectorSubcoreMesh(core_axis_name="c", subcore_axis_name="s"),
        out_type=jax.ShapeDtypeStruct((_BATCH, _SEQ, 128), jnp.float32),
        scratch_types=[
            pltpu.VMEM((_SEQ, _HIDDEN), jnp.float32),
            pltpu.VMEM((_IDXN,), jnp.int32),
            pltpu.VMEM((_IDXN,), jnp.int32),
            pltpu.VMEM((_SEQ, _HIDDEN), jnp.float32),
            pltpu.VMEM((_SEQ, _HIDDEN), jnp.float32),
            pltpu.VMEM((_SEQ, _HIDDEN), jnp.float32),
            pltpu.VMEM((_SEQ, _HIDDEN), jnp.float32),
            pltpu.SemaphoreType.DMA,
            pltpu.SemaphoreType.DMA,
            pltpu.SemaphoreType.DMA,
            pltpu.SemaphoreType.DMA,
            pltpu.SemaphoreType.DMA,
            pltpu.SemaphoreType.DMA,
        ],
        compiler_params=pltpu.CompilerParams(use_tc_tiling_on_sc=False),
    )
    return kfn(tokens_flat, emb_pad, pos_table)


def kernel(tokens, emb_table, pos_table):
    batch, seq = tokens.shape
    hid = emb_table.shape[1]
    emb_repack = _repack(emb_table.T)
    emb_lin = emb_repack.reshape(_VOCAB, _HIDDEN)
    out3 = _emb_lookup(tokens.reshape(-1), emb_lin, pos_table)
    return out3[:, :, :hid]


# speedup vs baseline: 2.0314x; 2.0314x over previous
import jax
import jax.numpy as jnp
import numpy as np
from jax import lax
from jax.experimental import pallas as pl
from jax.experimental.pallas import tpu as pltpu
from jax.experimental.pallas import tpu_sc as plsc

_VOCAB = 1000000
_HIDDEN = 64
_SEQ = 200
_BATCH = 4096

_NC = 2
_NS = 16
_NW = _NC * _NS
_SEQ_PER_W = _BATCH // _NW

_G0 = 128
_G1 = _SEQ - _G0
_UNROLL = 4
_IDXN = 208

_RB = 16384


_SUB = 128

_A_NP = np.zeros((2, _SUB // 2, _SUB), np.float32)
_A_NP[0, np.arange(_SUB // 2), 2 * np.arange(_SUB // 2)] = 1.0
_A_NP[1, np.arange(_SUB // 2), 2 * np.arange(_SUB // 2) + 1] = 1.0


def _repack_body(a_ref, in_ref, out_ref):
    x = in_ref[...]
    a = a_ref[...]
    a_e = a[: _SUB // 2]
    a_o = a[_SUB // 2:]
    dn = (((1,), (1,)), ((), ()))
    parts = []
    for k in range(_RB // _SUB):
        xs = x[:, k * _SUB:(k + 1) * _SUB]
        e = jax.lax.dot_general(a_e, xs, dn,
                                preferred_element_type=jnp.float32)
        o = jax.lax.dot_general(a_o, xs, dn,
                                preferred_element_type=jnp.float32)
        parts.append(jnp.concatenate([e, o], axis=1))
    out_ref[...] = jnp.concatenate(parts, axis=0)


def _repack(emb_t):
    grid = (pl.cdiv(_VOCAB, _RB),)
    return pl.pallas_call(
        _repack_body,
        grid=grid,
        in_specs=[pl.BlockSpec((_SUB, _SUB), lambda j: (0, 0)),
                  pl.BlockSpec((_HIDDEN, _RB), lambda j: (0, j))],
        out_specs=pl.BlockSpec((_RB // 2, 128), lambda j: (j, 0)),
        out_shape=jax.ShapeDtypeStruct((_VOCAB * _HIDDEN // 128, 128),
                                       jnp.float32),
    )(jnp.asarray(_A_NP.reshape(_SUB, _SUB)), emb_t)


def _body(tok_hbm, emb_hbm, pos_hbm, out_hbm, pos_v,
          idxr0, idxr1, rows0, rows1, wb0, wb1,
          si0, si1, sg0, sg1, so0, so1):
    c = lax.axis_index("c")
    s = lax.axis_index("s")
    wid = s * _NC + c

    idxr = (idxr0, idxr1)
    rows = (rows0, rows1)
    wb = (wb0, wb1)
    si = (si0, si1)
    sg = (sg0, sg1)
    so = (so0, so1)

    def fire_idx(i, b):
        base = (wid * _SEQ_PER_W + i) * _SEQ
        pltpu.async_copy(
            tok_hbm.at[pl.ds(base, _SEQ)], idxr[b].at[pl.ds(0, _SEQ)], si[b])

    def wait_idx(b):
        pltpu.make_async_copy(
            tok_hbm.at[pl.ds(0, _SEQ)], idxr[b].at[pl.ds(0, _SEQ)],
            si[b]).wait()

    def fire_gather(b):
        pltpu.async_copy(
            emb_hbm.at[idxr[b].at[pl.ds(0, _G0)]], rows[b].at[pl.ds(0, _G0)],
            sg[b])
        pltpu.async_copy(
            emb_hbm.at[idxr[b].at[pl.ds(_G0, _G1)]],
            rows[b].at[pl.ds(_G0, _G1)], sg[b])

    def wait_gather(b):
        pltpu.make_async_copy(
            emb_hbm.at[pl.ds(0, _SEQ)], rows[b], sg[b]).wait()

    def fire_wb(i, b):
        seq = wid * _SEQ_PER_W + i
        pltpu.async_copy(wb[b], out_hbm.at[seq, :, pl.ds(0, _HIDDEN)],
                         so[b])

    def wait_wb(b):
        pltpu.make_async_copy(
            out_hbm.at[0, :, pl.ds(0, _HIDDEN)], wb[b], so[b]).wait()

    pltpu.sync_copy(pos_hbm, pos_v)

    fire_idx(0, 0)
    fire_idx(1, 1)
    wait_idx(0)
    fire_gather(0)

    def seq_body(i, carry):
        b = lax.rem(i, 2)

        def for_buf(bb):
            nb = 1 - bb

            @pl.when(i + 1 < _SEQ_PER_W)
            def _():
                wait_idx(nb)
                fire_gather(nb)

            wait_gather(bb)

            @pl.when(i >= 2)
            def _():
                wait_wb(bb)

            def l_body(j, c2):
                for u in range(_UNROLL):
                    l = j * _UNROLL + u
                    for g in range(_HIDDEN // 16):
                        sl = pl.ds(g * 16, 16)
                        wb[bb][l, sl] = rows[bb][l, sl] + pos_v[l, sl]
                return c2

            lax.fori_loop(0, _SEQ // _UNROLL, l_body, 0)

            fire_wb(i, bb)

            @pl.when(i + 2 < _SEQ_PER_W)
            def _():
                fire_idx(i + 2, bb)

        @pl.when(b == 0)
        def _():
            for_buf(0)

        @pl.when(b == 1)
        def _():
            for_buf(1)

        return carry

    lax.fori_loop(0, _SEQ_PER_W, seq_body, 0)

    wait_wb((_SEQ_PER_W - 1) % 2)


def _emb_lookup(tokens_flat, emb_pad, pos_table):
    kfn = pl.kernel(
        _body,
        mesh=plsc.VectorSubcoreMesh(core_axis_name="c", subcore_axis_name="s"),
        out_type=jax.ShapeDtypeStruct((_BATCH, _SEQ, 128), jnp.float32),
        scratch_types=[
            pltpu.VMEM((_SEQ, _HIDDEN), jnp.float32),
            pltpu.VMEM((_IDXN,), jnp.int32),
            pltpu.VMEM((_IDXN,), jnp.int32),
            pltpu.VMEM((_SEQ, _HIDDEN), jnp.float32),
            pltpu.VMEM((_SEQ, _HIDDEN), jnp.float32),
            pltpu.VMEM((_SEQ, _HIDDEN), jnp.float32),
            pltpu.VMEM((_SEQ, _HIDDEN), jnp.float32),
            pltpu.SemaphoreType.DMA,
            pltpu.SemaphoreType.DMA,
            pltpu.SemaphoreType.DMA,
            pltpu.SemaphoreType.DMA,
            pltpu.SemaphoreType.DMA,
            pltpu.SemaphoreType.DMA,
        ],
        compiler_params=pltpu.CompilerParams(use_tc_tiling_on_sc=False),
    )
    return kfn(tokens_flat, emb_pad, pos_table)


def kernel(tokens, emb_table, pos_table):
    batch, seq = tokens.shape
    hid = emb_table.shape[1]
    emb_repack = _repack(emb_table.T)
    emb_lin = emb_repack.reshape(_VOCAB, _HIDDEN)
    out3 = _emb_lookup(tokens.reshape(-1), emb_lin, pos_table)
    return out3[:, :, :hid]
